# baseline (speedup 1.0000x reference)
import functools

import jax
import jax.numpy as jnp
from jax import lax
from jax.experimental import pallas as pl
from jax.experimental.pallas import tpu as pltpu
from jax.experimental.pallas import tpu_sc as plsc

NC = 2
NS = 16
NW = NC * NS
L = 16
SW = 4096


def _sigmoid(x):
    e = jnp.exp(-jnp.abs(x))
    return jnp.where(x >= 0, 1.0 / (1.0 + e), e / (1.0 + e))


def _tanh(x):
    e = jnp.exp(-2.0 * jnp.abs(x))
    t = (1.0 - e) / (1.0 + e)
    return jnp.where(x >= 0, t, -t)


@jax.jit
def kernel(inputs, state, gate_kernel_users, gate_kernel_items, gate_bias,
           candidate_kernel_users, candidate_kernel_items, candidate_bias):
    B, H = state.shape
    BPW = B // NW
    u_idx = inputs[:, 0].astype(jnp.int32)
    i_idx = inputs[:, 1].astype(jnp.int32)
    U1 = gate_kernel_users.shape[0]
    GT = gate_kernel_users.transpose(1, 2, 0).reshape(2 * H * H, U1)
    CT = candidate_kernel_users.transpose(1, 2, 0).reshape(H * H, U1)
    GIT = gate_kernel_items.T
    CIT = candidate_kernel_items.T
    ST = state.T
    NSEG = U1 // SW
    TOFF = NSEG * SW
    TWID = U1 - TOFF
    NF = 2 * H * H + 2 * H
    iota_b = jnp.arange(B, dtype=jnp.int32)
    su, sp = lax.sort([u_idx, iota_b], num_keys=1)
    si, ip = lax.sort([i_idx, iota_b], num_keys=1)
    edges = jnp.concatenate([jnp.arange(NSEG + 1, dtype=jnp.int32) * SW,
                             jnp.array([U1], jnp.int32)])
    ubnd = jnp.searchsorted(su, edges).astype(jnp.int32)
    ibnd = jnp.searchsorted(si, edges).astype(jnp.int32)
    NB = (NSEG + 2 + L - 1) // L * L
    ubnd = jnp.pad(ubnd, (0, NB - NSEG - 2))
    ibnd = jnp.pad(ibnd, (0, NB - NSEG - 2))

    mesh = plsc.VectorSubcoreMesh(
        core_axis_name="c", subcore_axis_name="s",
        num_cores=NC, num_subcores=NS)

    @functools.partial(
        pl.kernel,
        out_type=jax.ShapeDtypeStruct((NF, B), jnp.float32),
        mesh=mesh,
        scratch_types=[
            pltpu.VMEM((B,), jnp.int32),
            pltpu.VMEM((B,), jnp.int32),
            pltpu.VMEM((B,), jnp.int32),
            pltpu.VMEM((B,), jnp.int32),
            pltpu.VMEM((NB,), jnp.int32),
            pltpu.VMEM((NB,), jnp.int32),
            pltpu.VMEM((8, SW), jnp.float32),
            pltpu.VMEM((8, SW), jnp.float32),
            pltpu.VMEM((8, B + L), jnp.float32),
            pltpu.VMEM((1, 1697), jnp.float32),
            pltpu.VMEM((1, 1697), jnp.float32),
            pltpu.SemaphoreType.DMA,
            pltpu.SemaphoreType.DMA,
            pltpu.SemaphoreType.DMA,
            pltpu.SemaphoreType.DMA,
        ],
        compiler_params=pltpu.CompilerParams(needs_layout_passes=False),
    )
    def gather_rows(su_hbm, sp_hbm, si_hbm, ip_hbm, ub_hbm, ib_hbm,
                    gt_hbm, ct_hbm, git_hbm, cit_hbm, x_hbm,
                    suv, spv, siv, ipv, ubv, ibv, b0, b1, ob, tb0, tb1,
                    s0, s1, so, st):
        wid = lax.axis_index("s") * NC + lax.axis_index("c")
        pltpu.sync_copy(su_hbm, suv)
        pltpu.sync_copy(sp_hbm, spv)
        pltpu.sync_copy(si_hbm, siv)
        pltpu.sync_copy(ip_hbm, ipv)
        pltpu.sync_copy(ub_hbm, ubv)
        pltpu.sync_copy(ib_hbm, ibv)
        zero16 = jnp.zeros((L,), jnp.int32)
        bufs = (b0, b1)
        sems = (s0, s1)
        splat8 = [jnp.full((L,), j, jnp.int32) for j in range(8)]

        def segw(s):
            return SW if s < NSEG else TWID
        tbufs = (tb0, tb1)

        def bnd_at(bv, s):
            return bv[pl.ds(s // L * L, L)][s % L]

        def seg_copy(src_hbm, band0, nrows, s, bi):
            return pltpu.make_async_copy(
                src_hbm.at[pl.ds(band0, nrows), pl.ds(s * SW, segw(s))],
                bufs[bi].at[pl.ds(0, nrows), pl.ds(0, segw(s))], sems[bi])

        def seg_gather(idv, posv, bndv, s, bi, nrows, from_tail=False):
            lo = s * SW
            hi = lo + segw(s)
            srcbuf = tbufs[bi] if from_tail else bufs[bi]
            g_lo = bnd_at(bndv, s) // L
            g_hi = (bnd_at(bndv, s + 1) + L - 1) // L

            def body(g, carry):
                u = idv[pl.ds(g * L, L)]
                m = (u >= lo) & (u < hi)
                ul = u - lo
                pos = posv[pl.ds(g * L, L)]
                for j in range(nrows):
                    v = plsc.load_gather(srcbuf, [splat8[j], ul], mask=m)
                    plsc.store_scatter(ob, [splat8[j], pos], v, mask=m)
                return carry

            lax.fori_loop(g_lo, g_hi, body, 0)

        def tail_copy(src_hbm, row, tbi):
            return pltpu.make_async_copy(
                src_hbm.at[pl.ds(row, 1), pl.ds(TOFF, TWID)],
                tbufs[tbi], st)

        def band_tail(src_hbm, band0, idv, posv, bndv):
            lo = TOFF
            g_lo = bnd_at(bndv, NSEG) // L
            g_hi = (bnd_at(bndv, NSEG + 1) + L - 1) // L
            tail_copy(src_hbm, band0, 0).start()
            for j in range(8):
                if j + 1 < 8:
                    tail_copy(src_hbm, band0 + j + 1, (j + 1) % 2).start()
                tail_copy(src_hbm, band0 + j, j % 2).wait()

                def body(g, carry, j=j):
                    u = idv[pl.ds(g * L, L)]
                    m = (u >= lo) & (u < U1)
                    ul = u - lo
                    pos = posv[pl.ds(g * L, L)]
                    v = plsc.load_gather(tbufs[j % 2], [zero16, ul], mask=m)
                    plsc.store_scatter(ob, [splat8[j], pos], v, mask=m)
                    return carry

                lax.fori_loop(g_lo, g_hi, body, 0)

        def flush_band(out_base, nrows):
            cps = []
            for j in range(nrows):
                cps.append(pltpu.make_async_copy(
                    ob.at[pl.ds(j, 1), pl.ds(0, B)],
                    x_hbm.at[pl.ds(out_base + j, 1), :], so))
            return cps

        gband = (wid // 2) * 2 * H + H + (wid % 2) * 8
        bands = [
            (gt_hbm, gband, suv, spv, ubv, wid * 8),
            (ct_hbm, wid * 8, suv, spv, ubv, H * H + wid * 8),
        ]
        segs = [(b, s) for b in range(2) for s in range(NSEG)]
        for i in range(2):
            b, s = segs[i]
            seg_copy(bands[b][0], bands[b][1], 8, s, i).start()
        pending_flush = []
        for i, (b, s) in enumerate(segs):
            bi = i % 2
            src_hbm, band0, idv, posv, bndv, out_base = bands[b]
            seg_copy(src_hbm, band0, 8, s, bi).wait()
            if s == 0 and pending_flush:
                for cp in pending_flush:
                    cp.wait()
                pending_flush = []
            seg_gather(idv, posv, bndv, s, bi, 8)
            if i + 2 < len(segs):
                nb, ns = segs[i + 2]
                seg_copy(bands[nb][0], bands[nb][1], 8, ns, bi).start()
            if s == NSEG - 1:
                band_tail(src_hbm, band0, idv, posv, bndv)
                for cp in flush_band(out_base, 8):
                    cp.start()
                pending_flush = flush_band(out_base, 8)
        for cp in pending_flush:
            cp.wait()

        def item_seg_copy(src_hbm, src_row, s, bi):
            return pltpu.make_async_copy(
                src_hbm.at[pl.ds(src_row, 1), pl.ds(s * SW, segw(s))],
                bufs[bi].at[pl.ds(0, 1), pl.ds(0, segw(s))], sems[bi])

        def item_starts(s):
            @pl.when((wid < NS) & (s < NSEG + 1))
            def _():
                if s < NSEG:
                    item_seg_copy(git_hbm, H + wid, s, s % 2).start()
                else:
                    tail_copy(git_hbm, H + wid, 0).start()

            @pl.when((wid >= NS) & (s < NSEG + 1))
            def _():
                if s < NSEG:
                    item_seg_copy(cit_hbm, wid - NS, s, s % 2).start()
                else:
                    tail_copy(cit_hbm, wid - NS, 0).start()

        for t in range(2):
            item_starts(t)
        for s in range(NSEG):
            bi = s % 2
            item_seg_copy(git_hbm, 0, s, bi).wait()
            seg_gather(siv, ipv, ibv, s, bi, 1)
            item_starts(s + 2)
        tail_copy(git_hbm, 0, 0).wait()
        g_lo = bnd_at(ibv, NSEG) // L
        g_hi = (bnd_at(ibv, NSEG + 1) + L - 1) // L

        def tbody(g, carry):
            u = siv[pl.ds(g * L, L)]
            m = (u >= TOFF) & (u < U1)
            ul = u - TOFF
            pos = ipv[pl.ds(g * L, L)]
            v = plsc.load_gather(tbufs[0], [zero16, ul], mask=m)
            plsc.store_scatter(ob, [zero16, pos], v, mask=m)
            return carry

        lax.fori_loop(g_lo, g_hi, tbody, 0)
        pltpu.sync_copy(ob.at[pl.ds(0, 1), pl.ds(0, B)],
                        x_hbm.at[pl.ds(2 * H * H + wid, 1), :])

    @functools.partial(
        pl.kernel,
        out_type=jax.ShapeDtypeStruct((H, B), jnp.float32),
        mesh=mesh,
        scratch_types=[
            pltpu.VMEM((NF, BPW), jnp.float32),
            pltpu.VMEM((H, BPW), jnp.float32),
            pltpu.VMEM((2 * H,), jnp.float32),
            pltpu.VMEM((H,), jnp.float32),
            pltpu.VMEM((H, BPW), jnp.float32),
        ],
    )
    def compute(x_hbm, st_hbm, gb_hbm, cb_hbm, out_hbm,
                xv, sv, gbv, cbv, ov):
        wid = lax.axis_index("s") * NC + lax.axis_index("c")
        base = wid * BPW
        pltpu.sync_copy(x_hbm.at[:, pl.ds(base, BPW)], xv)
        pltpu.sync_copy(st_hbm.at[:, pl.ds(base, BPW)], sv)
        pltpu.sync_copy(gb_hbm, gbv)
        pltpu.sync_copy(cb_hbm, cbv)
        gbh = gbv[pl.ds(H, H)]
        cbh = cbv[...]

        def group(g, carry):
            gs = pl.ds(g * L, L)
            sh = [sv[h, gs] for h in range(H)]
            for k in range(H):
                acc_u = xv[2 * H * H + k, gs] + gbh[k]
                acc_c = xv[2 * H * H + H + k, gs] + cbh[k]
                for h in range(H):
                    acc_u = acc_u + sh[h] * xv[h * H + k, gs]
                    acc_c = acc_c + sh[h] * xv[H * H + h * H + k, gs]
                u_gate = _sigmoid(acc_u)
                c = _tanh(acc_c)
                ov[k, gs] = u_gate * sh[k] + (1.0 - u_gate) * c
            return carry

        lax.fori_loop(0, BPW // L, group, 0)
        pltpu.sync_copy(ov, out_hbm.at[:, pl.ds(base, BPW)])

    x = gather_rows(su, sp, si, ip, ubnd, ibnd, GT, CT, GIT, CIT)
    out = compute(x, ST, gate_bias, candidate_bias)
    return out.T


# speedup vs baseline: 1.2420x; 1.2420x over previous
import functools

import jax
import jax.numpy as jnp
from jax import lax
from jax.experimental import pallas as pl
from jax.experimental.pallas import tpu as pltpu
from jax.experimental.pallas import tpu_sc as plsc

NC = 2
NS = 16
NW = NC * NS
L = 16


def _sigmoid(x):
    e = jnp.exp(-jnp.abs(x))
    return jnp.where(x >= 0, 1.0 / (1.0 + e), e / (1.0 + e))


def _tanh(x):
    e = jnp.exp(-2.0 * jnp.abs(x))
    t = (1.0 - e) / (1.0 + e)
    return jnp.where(x >= 0, t, -t)


@jax.jit
def kernel(inputs, state, gate_kernel_users, gate_kernel_items, gate_bias,
           candidate_kernel_users, candidate_kernel_items, candidate_bias):
    B, H = state.shape
    BPW = B // NW
    NGALL = B // L
    u_idx = inputs[:, 0].astype(jnp.int32)
    i_idx = inputs[:, 1].astype(jnp.int32)
    U1 = gate_kernel_users.shape[0]
    GT = gate_kernel_users.transpose(1, 2, 0).reshape(2 * H * H, U1)
    CT = candidate_kernel_users.transpose(1, 2, 0).reshape(H * H, U1)
    GIT = gate_kernel_items.T
    CIT = candidate_kernel_items.T
    ST = state.T
    T0 = (U1 // 3) // 128 * 128
    OFFS = (0, T0, 2 * T0)
    TW = (T0, T0, U1 - 2 * T0)
    TMAX = max(TW)
    NF = 2 * H * H + 2 * H
    NRING = 16

    mesh = plsc.VectorSubcoreMesh(
        core_axis_name="c", subcore_axis_name="s",
        num_cores=NC, num_subcores=NS)

    @functools.partial(
        pl.kernel,
        out_type=jax.ShapeDtypeStruct((NF, B), jnp.float32),
        mesh=mesh,
        scratch_types=[
            pltpu.VMEM((B,), jnp.int32),
            pltpu.VMEM((B,), jnp.int32),
            pltpu.VMEM((1, TMAX), jnp.float32),
            pltpu.VMEM((1, TMAX), jnp.float32),
            pltpu.VMEM((1, TMAX), jnp.float32),
            pltpu.VMEM((1, B + L), jnp.float32),
            pltpu.VMEM((1, B + L), jnp.float32),
            pltpu.VMEM((B + 3 * L,), jnp.int32),
            pltpu.VMEM((B + 3 * L,), jnp.int32),
            pltpu.SemaphoreType.DMA,
            pltpu.SemaphoreType.DMA,
            pltpu.SemaphoreType.DMA,
            pltpu.SemaphoreType.DMA,
        ],
        compiler_params=pltpu.CompilerParams(needs_layout_passes=False),
    )
    def gather_rows(u_hbm, i_hbm, gt_hbm, ct_hbm, git_hbm, cit_hbm, x_hbm,
                    uv, iv, b0, b1, b2, oe, oo, lid, lpos, s0, s1, s2, so):
        wid = lax.axis_index("s") * NC + lax.axis_index("c")
        pltpu.sync_copy(u_hbm, uv)
        pltpu.sync_copy(i_hbm, iv)
        zero16 = jnp.zeros((L,), jnp.int32)
        iota16 = lax.iota(jnp.int32, L)
        bufs = (b0, b1, b2)
        sems = (s0, s1, s2)
        orows = (oe, oo)

        parts = []
        off = jnp.int32(0)
        for t in range(3):
            lo, hi = OFFS[t], OFFS[t] + TW[t]
            start = off

            def build(g, o, lo=lo, hi=hi):
                u = uv[pl.ds(g * L, L)]
                m = (u >= lo) & (u < hi)
                plsc.store_compressed(lid.at[pl.ds(o, L)], u - lo, mask=m)
                plsc.store_compressed(lpos.at[pl.ds(o, L)],
                                      g * L + iota16, mask=m)
                return o + plsc.all_reduce_population_count(m)[0]

            off = lax.fori_loop(0, NGALL, build, off)
            cnt = off - start
            lid[pl.ds(off, L)] = jnp.zeros((L,), jnp.int32)
            lpos[pl.ds(off, L)] = jnp.full((L,), B, jnp.int32)
            off = off + L
            parts.append((start, (cnt + L - 1) // L))

        def seg_copy(src_hbm, src_row, t, bi):
            return pltpu.make_async_copy(
                src_hbm.at[pl.ds(src_row, 1), pl.ds(OFFS[t], TW[t])],
                bufs[bi].at[:, pl.ds(0, TW[t])], sems[bi])

        def seg_gather(ids_v, t, bi, orow):
            lo, hi = OFFS[t], OFFS[t] + TW[t]

            def body(g, carry):
                u = ids_v[pl.ds(g * L, L)]
                m = (u >= lo) & (u < hi)
                v = plsc.load_gather(bufs[bi], [zero16, u - lo], mask=m)
                if t == 0:
                    orow[0, pl.ds(g * L, L)] = v
                else:
                    plsc.store_scatter(orow, [zero16, g * L + iota16], v,
                                       mask=m)
                return carry

            lax.fori_loop(0, NGALL, body, 0)

        def seg_gather_packed(t, bi, orow):
            start, ng = parts[t]

            def body(g, carry):
                o = start + g * L
                ul = lid[pl.ds(o, L)]
                pos = lpos[pl.ds(o, L)]
                v = plsc.load_gather(bufs[bi], [zero16, ul])
                plsc.store_scatter(orow, [zero16, pos], v)
                return carry

            lax.fori_loop(0, ng, body, 0)

        def out_copy(orow, out_row):
            return pltpu.make_async_copy(
                orow.at[:, pl.ds(0, B)], x_hbm.at[pl.ds(out_row, 1), :], so)

        rows = []
        for j in range(8):
            r = wid * 8 + j
            h = r // H
            k = lax.rem(r, H)
            rows.append((gt_hbm, h * 2 * H + H + k, r))
        for j in range(8):
            r = wid * 8 + j
            rows.append((ct_hbm, r, H * H + r))

        segs = [(ri, t) for ri in range(NRING) for t in range(3)]
        for i in range(3):
            ri, t = segs[i]
            seg_copy(rows[ri][0], rows[ri][1], t, i).start()
        for i, (ri, t) in enumerate(segs):
            bi = i % 3
            src_hbm, src_row, out_row = rows[ri]
            seg_copy(src_hbm, src_row, t, bi).wait()
            if t == 0 and ri >= 2:
                out_copy(orows[ri % 2], rows[ri - 2][2]).wait()
            seg_gather_packed(t, bi, orows[ri % 2])
            if i + 3 < len(segs):
                nri, nt = segs[i + 3]
                seg_copy(rows[nri][0], rows[nri][1], nt, bi).start()
            if t == 2:
                out_copy(orows[ri % 2], out_row).start()
        out_copy(orows[0], rows[NRING - 2][2]).wait()
        out_copy(orows[1], rows[NRING - 1][2]).wait()

        @pl.when(wid < NS)
        def _():
            for t in range(3):
                seg_copy(git_hbm, H + wid, t, t).start()
            for t in range(3):
                seg_copy(git_hbm, H + wid, t, t).wait()
                seg_gather(iv, t, t, oe)
            pltpu.sync_copy(oe.at[:, pl.ds(0, B)],
                            x_hbm.at[pl.ds(2 * H * H + wid, 1), :])

        @pl.when(wid >= NS)
        def _():
            for t in range(3):
                seg_copy(cit_hbm, wid - NS, t, t).start()
            for t in range(3):
                seg_copy(cit_hbm, wid - NS, t, t).wait()
                seg_gather(iv, t, t, oe)
            pltpu.sync_copy(
                oe.at[:, pl.ds(0, B)],
                x_hbm.at[pl.ds(2 * H * H + H + wid - NS, 1), :])

    @functools.partial(
        pl.kernel,
        out_type=jax.ShapeDtypeStruct((H, B), jnp.float32),
        mesh=mesh,
        scratch_types=[
            pltpu.VMEM((NF, BPW), jnp.float32),
            pltpu.VMEM((H, BPW), jnp.float32),
            pltpu.VMEM((2 * H,), jnp.float32),
            pltpu.VMEM((H,), jnp.float32),
            pltpu.VMEM((H, BPW), jnp.float32),
        ],
    )
    def compute(x_hbm, st_hbm, gb_hbm, cb_hbm, out_hbm,
                xv, sv, gbv, cbv, ov):
        wid = lax.axis_index("s") * NC + lax.axis_index("c")
        base = wid * BPW
        pltpu.sync_copy(x_hbm.at[:, pl.ds(base, BPW)], xv)
        pltpu.sync_copy(st_hbm.at[:, pl.ds(base, BPW)], sv)
        pltpu.sync_copy(gb_hbm, gbv)
        pltpu.sync_copy(cb_hbm, cbv)
        gbh = gbv[pl.ds(H, H)]
        cbh = cbv[...]

        def group(g, carry):
            gs = pl.ds(g * L, L)
            sh = [sv[h, gs] for h in range(H)]
            for k in range(H):
                acc_u = xv[2 * H * H + k, gs] + gbh[k]
                acc_c = xv[2 * H * H + H + k, gs] + cbh[k]
                for h in range(H):
                    acc_u = acc_u + sh[h] * xv[h * H + k, gs]
                    acc_c = acc_c + sh[h] * xv[H * H + h * H + k, gs]
                u_gate = _sigmoid(acc_u)
                c = _tanh(acc_c)
                ov[k, gs] = u_gate * sh[k] + (1.0 - u_gate) * c
            return carry

        lax.fori_loop(0, BPW // L, group, 0)
        pltpu.sync_copy(ov, out_hbm.at[:, pl.ds(base, BPW)])

    x = gather_rows(u_idx, i_idx, GT, CT, GIT, CIT)
    out = compute(x, ST, gate_bias, candidate_bias)
    return out.T
